# baseline (speedup 1.0000x reference)
import jax
import jax.numpy as jnp
from jax.experimental import pallas as pl
from jax.experimental.pallas import tpu as pltpu

NUM_EXPERTS = 16
TOP_K = 2
BLOCK_T = 1024
BLOCK_D = 512


def _router_block(x_ref, w_ref, b_ref, o_ref):
    j = pl.program_id(1)
    nd = pl.num_programs(1)
    part = jnp.dot(x_ref[...], w_ref[...], preferred_element_type=jnp.float32)

    @pl.when(j == 0)
    def _init():
        o_ref[...] = part + b_ref[...]

    @pl.when(j != 0)
    def _acc():
        o_ref[...] += part

    @pl.when(j == nd - 1)
    def _tail():
        logits = o_ref[...]
        m = jnp.max(logits, axis=-1, keepdims=True)
        e = jnp.exp(logits - m)
        p = e * (1.0 / jnp.sum(e, axis=-1, keepdims=True))
        ii = jax.lax.broadcasted_iota(jnp.int32, logits.shape, 1)
        i1 = jnp.argmax(logits, axis=-1, keepdims=True)
        sel1 = ii == i1
        i2 = jnp.argmax(jnp.where(sel1, -jnp.inf, logits), axis=-1, keepdims=True)
        mask = sel1 | (ii == i2)
        o_ref[...] = jnp.where(mask, p, 0.0)


def kernel(token_inputs, W, b, num_experts):
    B, S, D = token_inputs.shape
    E = W.shape[1]
    x = token_inputs.reshape(B * S, D)
    b2 = b.reshape(1, E)
    grid = (B * S // BLOCK_T, D // BLOCK_D)
    out = pl.pallas_call(
        _router_block,
        grid=grid,
        in_specs=[
            pl.BlockSpec((BLOCK_T, BLOCK_D), lambda i, j: (i, j)),
            pl.BlockSpec((BLOCK_D, E), lambda i, j: (j, 0)),
            pl.BlockSpec((1, E), lambda i, j: (0, 0)),
        ],
        out_specs=pl.BlockSpec((BLOCK_T, E), lambda i, j: (i, 0)),
        out_shape=jax.ShapeDtypeStruct((B * S, E), jnp.float32),
        compiler_params=pltpu.CompilerParams(
            dimension_semantics=("parallel", "arbitrary"),
        ),
    )(x, W, b2)
    return out.reshape(B, S, E)


# speedup vs baseline: 1.3750x; 1.3750x over previous
import jax
import jax.numpy as jnp
from jax.experimental import pallas as pl
from jax.experimental.pallas import tpu as pltpu

NUM_EXPERTS = 16
TOP_K = 2
BLOCK_T = 1024


def _router_block(x_ref, w_ref, b_ref, o_ref, acc_ref):
    i = pl.program_id(0)
    nb = pl.num_programs(0)

    @pl.when(i < nb - 1)
    def _dot():
        part = jnp.dot(x_ref[...], w_ref[...], preferred_element_type=jnp.float32)
        acc_ref[i % 2] = part + b_ref[...]

    @pl.when(i > 0)
    def _tail():
        logits = acc_ref[(i - 1) % 2]
        m = jnp.max(logits, axis=-1, keepdims=True)
        e = jnp.exp(logits - m)
        p = e * (1.0 / jnp.sum(e, axis=-1, keepdims=True))
        ii = jax.lax.broadcasted_iota(jnp.int32, logits.shape, 1)
        i1 = jnp.argmax(logits, axis=-1, keepdims=True)
        sel1 = ii == i1
        i2 = jnp.argmax(jnp.where(sel1, -jnp.inf, logits), axis=-1, keepdims=True)
        mask = sel1 | (ii == i2)
        o_ref[...] = jnp.where(mask, p, 0.0)


def kernel(token_inputs, W, b, num_experts):
    B, S, D = token_inputs.shape
    E = W.shape[1]
    x = token_inputs.reshape(B * S, D)
    b2 = b.reshape(1, E)
    nb = B * S // BLOCK_T
    out = pl.pallas_call(
        _router_block,
        grid=(nb + 1,),
        in_specs=[
            pl.BlockSpec((BLOCK_T, D), lambda i: (jnp.minimum(i, nb - 1), 0)),
            pl.BlockSpec((D, E), lambda i: (0, 0)),
            pl.BlockSpec((1, E), lambda i: (0, 0)),
        ],
        out_specs=pl.BlockSpec((BLOCK_T, E), lambda i: (jnp.maximum(i - 1, 0), 0)),
        out_shape=jax.ShapeDtypeStruct((B * S, E), jnp.float32),
        scratch_shapes=[pltpu.VMEM((2, BLOCK_T, E), jnp.float32)],
        compiler_params=pltpu.CompilerParams(
            dimension_semantics=("arbitrary",),
        ),
    )(x, W, b2)
    return out.reshape(B, S, E)
